# baseline (speedup 1.0000x reference)
import functools

import jax
import jax.numpy as jnp
from jax.experimental import pallas as pl
from jax.experimental.pallas import tpu as pltpu

N, C, H, W = 4, 256, 64, 64
P = H * W
PPAD = P + 2 * W
A = 9
BR = 512
R = P // BR
TC = 512


def _rpn_kernel(x_ref, w_ref, b_ref, dw_ref, db_ref, o_ref,
                xp_ref, xl_ref, xr_ref):
    r = pl.program_id(1)

    @pl.when(r == 0)
    def _build_layout():
        zrow = jnp.zeros((W, C), dtype=jnp.bfloat16)
        xp_ref[pl.ds(0, W), :] = zrow
        xp_ref[pl.ds(W + P, W), :] = zrow
        for j in range(P // TC):
            chunk = x_ref[0, :, pl.ds(j * TC, TC)].astype(jnp.bfloat16)
            xp_ref[pl.ds(W + j * TC, TC), :] = chunk.T
        xc = xp_ref[...]
        col = jax.lax.broadcasted_iota(jnp.int32, (PPAD, C), 0) % W
        zero = jnp.zeros((), jnp.bfloat16)
        xl = pltpu.roll(xc, PPAD - 1, 0)
        xl_ref[...] = jnp.where(col != (W - 1), xl, zero)
        xr = pltpu.roll(xc, 1, 0)
        xr_ref[...] = jnp.where(col != 0, xr, zero)

    acc = jnp.zeros((BR, C), dtype=jnp.float32)
    base = W + r * BR
    for dy in (-1, 0, 1):
        start = base + dy * W
        for dx, src in ((-1, xr_ref), (0, xp_ref), (1, xl_ref)):
            k = (dy + 1) * 3 + (dx + 1)
            blk = src[pl.ds(start, BR), :]
            acc += jnp.dot(blk, w_ref[k], preferred_element_type=jnp.float32)
    t = jax.nn.relu(acc + b_ref[0]).astype(jnp.bfloat16)
    s = jnp.dot(t, dw_ref[...], preferred_element_type=jnp.float32) + db_ref[0]
    o_ref[0] = jax.nn.sigmoid(s)


@functools.partial(jax.jit, static_argnames=())
def kernel(x, img_shape, conv1_w, conv1_b, score_w, score_b, offset_w, offset_b):
    n = x.shape[0]
    xf = x.reshape(n, C, P)
    wr = jnp.transpose(conv1_w, (2, 3, 1, 0)).reshape(9, C, C).astype(jnp.bfloat16)
    b2 = conv1_b.reshape(1, C)
    sw = score_w[:, :, 0, 0]
    dw = (sw[1::2] - sw[0::2]).T.astype(jnp.bfloat16)
    db = (score_b[1::2] - score_b[0::2]).reshape(1, A)

    fg = pl.pallas_call(
        _rpn_kernel,
        grid=(n, R),
        in_specs=[
            pl.BlockSpec((1, C, P), lambda i, r: (i, 0, 0)),
            pl.BlockSpec((9, C, C), lambda i, r: (0, 0, 0)),
            pl.BlockSpec((1, C), lambda i, r: (0, 0)),
            pl.BlockSpec((C, A), lambda i, r: (0, 0)),
            pl.BlockSpec((1, A), lambda i, r: (0, 0)),
        ],
        out_specs=pl.BlockSpec((1, BR, A), lambda i, r: (i, r, 0)),
        out_shape=jax.ShapeDtypeStruct((n, P, A), jnp.float32),
        scratch_shapes=[
            pltpu.VMEM((PPAD, C), jnp.bfloat16),
            pltpu.VMEM((PPAD, C), jnp.bfloat16),
            pltpu.VMEM((PPAD, C), jnp.bfloat16),
        ],
    )(xf, wr, b2, dw, db)

    return fg.reshape(n, P * A // 2, 2)


# speedup vs baseline: 1.0009x; 1.0009x over previous
import functools

import jax
import jax.numpy as jnp
from jax.experimental import pallas as pl
from jax.experimental.pallas import tpu as pltpu

N, C, H, W = 4, 256, 64, 64
P = H * W
PPAD = P + 2 * W
A = 9
BR = 512
R = P // BR
TC = 512


def _rpn_kernel(x_ref, w_ref, b_ref, dw_ref, db_ref, o_ref, xcat_ref):
    r = pl.program_id(1)

    @pl.when(r == 0)
    def _build_layout():
        zrow = jnp.zeros((W, 3 * C), dtype=jnp.bfloat16)
        xcat_ref[pl.ds(0, W), :] = zrow
        xcat_ref[pl.ds(W + P, W), :] = zrow
        for j in range(P // TC):
            chunk = x_ref[0, :, pl.ds(j * TC, TC)].astype(jnp.bfloat16)
            xcat_ref[pl.ds(W + j * TC, TC), C:2 * C] = chunk.T
        xc = xcat_ref[:, C:2 * C]
        col = jax.lax.broadcasted_iota(jnp.int32, (PPAD, C), 0) % W
        zero = jnp.zeros((), jnp.bfloat16)
        xr = pltpu.roll(xc, 1, 0)
        xcat_ref[:, 0:C] = jnp.where(col != 0, xr, zero)
        xl = pltpu.roll(xc, PPAD - 1, 0)
        xcat_ref[:, 2 * C:3 * C] = jnp.where(col != (W - 1), xl, zero)

    acc = jnp.zeros((BR, C), dtype=jnp.float32)
    base = W + r * BR
    for dy in (-1, 0, 1):
        blk = xcat_ref[pl.ds(base + dy * W, BR), :]
        acc += jnp.dot(blk, w_ref[dy + 1],
                       preferred_element_type=jnp.float32)
    t = jax.nn.relu(acc + b_ref[0]).astype(jnp.bfloat16)
    s = jnp.dot(t, dw_ref[...], preferred_element_type=jnp.float32) + db_ref[0]
    o_ref[0] = jax.nn.sigmoid(s)


@functools.partial(jax.jit, static_argnames=())
def kernel(x, img_shape, conv1_w, conv1_b, score_w, score_b, offset_w, offset_b):
    n = x.shape[0]
    xf = x.reshape(n, C, P)
    wr = jnp.transpose(conv1_w, (2, 3, 1, 0)).reshape(3, 3 * C, C)
    wr = wr.astype(jnp.bfloat16)
    b2 = conv1_b.reshape(1, C)
    sw = score_w[:, :, 0, 0]
    dw = (sw[1::2] - sw[0::2]).T.astype(jnp.bfloat16)
    db = (score_b[1::2] - score_b[0::2]).reshape(1, A)

    fg = pl.pallas_call(
        _rpn_kernel,
        grid=(n, R),
        in_specs=[
            pl.BlockSpec((1, C, P), lambda i, r: (i, 0, 0)),
            pl.BlockSpec((3, 3 * C, C), lambda i, r: (0, 0, 0)),
            pl.BlockSpec((1, C), lambda i, r: (0, 0)),
            pl.BlockSpec((C, A), lambda i, r: (0, 0)),
            pl.BlockSpec((1, A), lambda i, r: (0, 0)),
        ],
        out_specs=pl.BlockSpec((1, BR, A), lambda i, r: (i, r, 0)),
        out_shape=jax.ShapeDtypeStruct((n, P, A), jnp.float32),
        scratch_shapes=[
            pltpu.VMEM((PPAD, 3 * C), jnp.bfloat16),
        ],
    )(xf, wr, b2, dw, db)

    return fg.reshape(n, P * A // 2, 2)


# speedup vs baseline: 1.1245x; 1.1235x over previous
import functools

import jax
import jax.numpy as jnp
from jax.experimental import pallas as pl
from jax.experimental.pallas import tpu as pltpu

N, C, H, W = 4, 256, 64, 64
P = H * W
PPAD = P + 2 * W
A = 9
BR = 2048
R = P // BR
TC = 512


def _rpn_kernel(x_ref, w_ref, b_ref, dw_ref, db_ref, o_ref, xcat_ref):
    r = pl.program_id(1)

    @pl.when(r == 0)
    def _build_layout():
        zrow = jnp.zeros((W, 3 * C), dtype=jnp.bfloat16)
        xcat_ref[pl.ds(0, W), :] = zrow
        xcat_ref[pl.ds(W + P, W), :] = zrow
        for j in range(P // TC):
            chunk = x_ref[0, :, pl.ds(j * TC, TC)].astype(jnp.bfloat16)
            xcat_ref[pl.ds(W + j * TC, TC), C:2 * C] = chunk.T
        xc = xcat_ref[:, C:2 * C]
        col = jax.lax.broadcasted_iota(jnp.int32, (PPAD, C), 0) % W
        zero = jnp.zeros((), jnp.bfloat16)
        xr = pltpu.roll(xc, 1, 0)
        xcat_ref[:, 0:C] = jnp.where(col != 0, xr, zero)
        xl = pltpu.roll(xc, PPAD - 1, 0)
        xcat_ref[:, 2 * C:3 * C] = jnp.where(col != (W - 1), xl, zero)

    acc = jnp.zeros((BR, C), dtype=jnp.float32)
    base = W + r * BR
    for dy in (-1, 0, 1):
        blk = xcat_ref[pl.ds(base + dy * W, BR), :]
        acc += jnp.dot(blk, w_ref[dy + 1],
                       preferred_element_type=jnp.float32)
    t = jax.nn.relu(acc + b_ref[0]).astype(jnp.bfloat16)
    s = jnp.dot(t, dw_ref[...], preferred_element_type=jnp.float32) + db_ref[0]
    o_ref[0] = jax.nn.sigmoid(s)


@functools.partial(jax.jit, static_argnames=())
def kernel(x, img_shape, conv1_w, conv1_b, score_w, score_b, offset_w, offset_b):
    n = x.shape[0]
    xf = x.reshape(n, C, P)
    wr = jnp.transpose(conv1_w, (2, 3, 1, 0)).reshape(3, 3 * C, C)
    wr = wr.astype(jnp.bfloat16)
    b2 = conv1_b.reshape(1, C)
    sw = score_w[:, :, 0, 0]
    dw = (sw[1::2] - sw[0::2]).T.astype(jnp.bfloat16)
    db = (score_b[1::2] - score_b[0::2]).reshape(1, A)

    fg = pl.pallas_call(
        _rpn_kernel,
        grid=(n, R),
        in_specs=[
            pl.BlockSpec((1, C, P), lambda i, r: (i, 0, 0)),
            pl.BlockSpec((3, 3 * C, C), lambda i, r: (0, 0, 0)),
            pl.BlockSpec((1, C), lambda i, r: (0, 0)),
            pl.BlockSpec((C, A), lambda i, r: (0, 0)),
            pl.BlockSpec((1, A), lambda i, r: (0, 0)),
        ],
        out_specs=pl.BlockSpec((1, BR, A), lambda i, r: (i, r, 0)),
        out_shape=jax.ShapeDtypeStruct((n, P, A), jnp.float32),
        scratch_shapes=[
            pltpu.VMEM((PPAD, 3 * C), jnp.bfloat16),
        ],
    )(xf, wr, b2, dw, db)

    return fg.reshape(n, P * A // 2, 2)
